# baseline (speedup 1.0000x reference)
import functools

import jax
import jax.numpy as jnp
from jax import lax
from jax.experimental import pallas as pl
from jax.experimental.pallas import tpu as pltpu
from jax.experimental.pallas import tpu_sc as plsc

_B, _C, _H, _W = 8, 256, 32, 32
_HW = _H * _W
_N = _B * _HW
_K = 8192

_TK = 1024
_KT = _K // _TK

_NC, _NS = 2, 16
_NW = _NC * _NS
_RPW = _N // _NW


def _tree_min_argmin(d, stop=1, ix=None):
    v = d
    size = d.shape[0]
    while size > stop:
        h = size // 2
        a, b = v[:h], v[h:]
        cond = b < a
        v = jnp.where(cond, b, a)
        if ix is None:
            ix = jnp.where(cond, jnp.int32(h), jnp.int32(0))
        else:
            ix = jnp.where(cond, ix[h:] + jnp.int32(h), ix[:h])
        size = h
    return v, ix


_KT2 = _KT // 2
_TS = _B * _KT2 + 1


_CH = 32


def _chunked_argmin(xsq, wsq, m_of):
    rv = ri = None
    for c0 in range(0, _TK, _CH):
        dc = (xsq + wsq[c0:c0 + _CH]) - m_of(c0)
        v, ix = _tree_min_argmin(dc, stop=8)
        ix = ix + jnp.int32(c0)
        if rv is None:
            rv, ri = v, ix
        else:
            cond = v < rv
            rv = jnp.where(cond, v, rv)
            ri = jnp.where(cond, ix, ri)
    return _tree_min_argmin(rv, stop=1, ix=ri)


def _argmin_body(x_ref, w_ref, idx_ref, loss_ref, m_s, wsq_s, xsq_s, bestv, besti):
    t = pl.program_id(0)
    j2 = t % _KT2

    pwsq = wsq_s[...]
    pxsq = xsq_s[...]

    xb = x_ref[0]
    wt01 = w_ref[...]
    wt0 = wt01[:_TK]
    wt1 = wt01[_TK:]

    m2a = jnp.dot(wt0 + wt0, xb, preferred_element_type=jnp.float32)

    dminA, drowA = _chunked_argmin(pxsq, pwsq, lambda c0: m_s[pl.ds(c0, _CH)])
    liA = drowA + ((2 * j2 - 1) % _KT) * _TK

    m2b = jnp.dot(wt1 + wt1, xb, preferred_element_type=jnp.float32)

    xsq = jnp.sum(xb * xb, axis=0, keepdims=True)
    wsq0 = jnp.sum(wt0 * wt0, axis=1, keepdims=True)
    wsq1 = jnp.sum(wt1 * wt1, axis=1, keepdims=True)

    dminB, drowB = _chunked_argmin(xsq, wsq0, lambda c0: m2a[c0:c0 + _CH])
    liB = drowB + (2 * j2) * _TK

    m_s[...] = m2b
    wsq_s[...] = wsq1
    xsq_s[...] = xsq

    @pl.when(t > 0)
    def _update_a():
        pv = bestv[...]
        upd = dminA < pv
        bestv[...] = jnp.where(upd, dminA, pv)
        besti[...] = jnp.where(upd, liA, besti[...])

    @pl.when((t > 0) & (j2 == 0))
    def _finalize():
        idx_ref[0] = besti[...]
        s = jnp.sum(bestv[...]) * (1.0 / (_N * _C))

        @pl.when(t == _KT2)
        def _():
            loss_ref[0, 0] = s

        @pl.when(t > _KT2)
        def _():
            loss_ref[0, 0] += s

    @pl.when(j2 == 0)
    def _init_b():
        bestv[...] = dminB
        besti[...] = liB

    @pl.when(j2 != 0)
    def _update_b():
        pv = bestv[...]
        upd = dminB < pv
        bestv[...] = jnp.where(upd, dminB, pv)
        besti[...] = jnp.where(upd, liB, besti[...])


_argmin_call = pl.pallas_call(
    _argmin_body,
    grid=(_TS,),
    in_specs=[
        pl.BlockSpec((1, _C, _HW), lambda t: (jnp.minimum(t // _KT2, _B - 1), 0, 0)),
        pl.BlockSpec(
            (2 * _TK, _C), lambda t: (jnp.minimum(t, _B * _KT2 - 1) % _KT2, 0)
        ),
    ],
    out_specs=[
        pl.BlockSpec((1, 1, _HW), lambda t: (jnp.maximum(t // _KT2 - 1, 0), 0, 0)),
        pl.BlockSpec((1, 1), lambda t: (0, 0), memory_space=pltpu.SMEM),
    ],
    out_shape=[
        jax.ShapeDtypeStruct((_B, 1, _HW), jnp.int32),
        jax.ShapeDtypeStruct((1, 1), jnp.float32),
    ],
    scratch_shapes=[
        pltpu.VMEM((_TK, _HW), jnp.float32),
        pltpu.VMEM((_TK, 1), jnp.float32),
        pltpu.VMEM((1, _HW), jnp.float32),
        pltpu.VMEM((1, _HW), jnp.float32),
        pltpu.VMEM((1, _HW), jnp.int32),
    ],
    compiler_params=pltpu.CompilerParams(
        dimension_semantics=("arbitrary",),
    ),
)


@functools.cache
def _gather_rows_call():
    @functools.partial(
        pl.kernel,
        out_type=jax.ShapeDtypeStruct((_N, _C), jnp.float32),
        mesh=plsc.VectorSubcoreMesh(core_axis_name="c", subcore_axis_name="s"),
        scratch_types=[
            pltpu.VMEM((_RPW,), jnp.int32),
            pltpu.VMEM((_RPW, _C), jnp.float32),
            pltpu.SemaphoreType.DMA,
        ],
    )
    def _gather_rows(w_hbm, idx_hbm, z_hbm, idx_v, rows_v, sem):
        wid = lax.axis_index("s") * _NC + lax.axis_index("c")
        base = wid * _RPW
        pltpu.sync_copy(idx_hbm.at[pl.ds(base, _RPW)], idx_v)
        pltpu.async_copy(w_hbm.at[idx_v], rows_v, sem).wait()
        pltpu.sync_copy(rows_v, z_hbm.at[pl.ds(base, _RPW)])

    return _gather_rows


def kernel(x, W):
    B, C, H, Wd = x.shape
    xr = x.reshape(B, C, H * Wd)
    midx, loss_sum = _argmin_call(xr, W)
    idx_flat = midx.reshape(B * H * Wd)
    z = _gather_rows_call()(W, idx_flat)
    z_out = jnp.transpose(z.reshape(B, H, Wd, C), (0, 3, 1, 2))
    sequence = midx.reshape(B, H, Wd)
    loss = loss_sum[0, 0]
    return (z_out, sequence, loss, loss)


# speedup vs baseline: 1.0465x; 1.0465x over previous
import functools

import jax
import jax.numpy as jnp
from jax import lax
from jax.experimental import pallas as pl
from jax.experimental.pallas import tpu as pltpu
from jax.experimental.pallas import tpu_sc as plsc

_B, _C, _H, _W = 8, 256, 32, 32
_HW = _H * _W
_N = _B * _HW
_K = 8192

_TK = 2048
_KT = _K // _TK

_NC, _NS = 2, 16
_NW = _NC * _NS
_RPW = _N // _NW


def _tree_min_argmin(d, stop=1, ix=None):
    v = d
    size = d.shape[0]
    while size > stop:
        h = size // 2
        a, b = v[:h], v[h:]
        cond = b < a
        v = jnp.where(cond, b, a)
        if ix is None:
            ix = jnp.where(cond, jnp.int32(h), jnp.int32(0))
        else:
            ix = jnp.where(cond, ix[h:] + jnp.int32(h), ix[:h])
        size = h
    return v, ix


_KT2 = _KT // 2
_TS = _B * _KT2 + 1


_CH = 32


def _chunked_argmin(xsq, wsq, m_of):
    rv = ri = None
    for c0 in range(0, _TK, _CH):
        dc = (xsq + wsq[c0:c0 + _CH]) - m_of(c0)
        v, ix = _tree_min_argmin(dc, stop=8)
        ix = ix + jnp.int32(c0)
        if rv is None:
            rv, ri = v, ix
        else:
            cond = v < rv
            rv = jnp.where(cond, v, rv)
            ri = jnp.where(cond, ix, ri)
    return _tree_min_argmin(rv, stop=1, ix=ri)


def _argmin_body(x_ref, w_ref, idx_ref, loss_ref, m_s, wsq_s, xsq_s, bestv, besti):
    t = pl.program_id(0)
    j2 = t % _KT2

    pwsq = wsq_s[...]
    pxsq = xsq_s[...]

    xb = x_ref[0]
    wt01 = w_ref[...]
    wt0 = wt01[:_TK]
    wt1 = wt01[_TK:]

    m2a = jnp.dot(wt0 + wt0, xb, preferred_element_type=jnp.float32)

    dminA, drowA = _chunked_argmin(pxsq, pwsq, lambda c0: m_s[pl.ds(c0, _CH)])
    liA = drowA + ((2 * j2 - 1) % _KT) * _TK

    m2b = jnp.dot(wt1 + wt1, xb, preferred_element_type=jnp.float32)

    xsq = jnp.sum(xb * xb, axis=0, keepdims=True)
    wsq0 = jnp.sum(wt0 * wt0, axis=1, keepdims=True)
    wsq1 = jnp.sum(wt1 * wt1, axis=1, keepdims=True)

    dminB, drowB = _chunked_argmin(xsq, wsq0, lambda c0: m2a[c0:c0 + _CH])
    liB = drowB + (2 * j2) * _TK

    m_s[...] = m2b
    wsq_s[...] = wsq1
    xsq_s[...] = xsq

    @pl.when(t > 0)
    def _update_a():
        pv = bestv[...]
        upd = dminA < pv
        bestv[...] = jnp.where(upd, dminA, pv)
        besti[...] = jnp.where(upd, liA, besti[...])

    @pl.when((t > 0) & (j2 == 0))
    def _finalize():
        idx_ref[0] = besti[...]
        s = jnp.sum(bestv[...]) * (1.0 / (_N * _C))

        @pl.when(t == _KT2)
        def _():
            loss_ref[0, 0] = s

        @pl.when(t > _KT2)
        def _():
            loss_ref[0, 0] += s

    @pl.when(j2 == 0)
    def _init_b():
        bestv[...] = dminB
        besti[...] = liB

    @pl.when(j2 != 0)
    def _update_b():
        pv = bestv[...]
        upd = dminB < pv
        bestv[...] = jnp.where(upd, dminB, pv)
        besti[...] = jnp.where(upd, liB, besti[...])


_argmin_call = pl.pallas_call(
    _argmin_body,
    grid=(_TS,),
    in_specs=[
        pl.BlockSpec((1, _C, _HW), lambda t: (jnp.minimum(t // _KT2, _B - 1), 0, 0)),
        pl.BlockSpec(
            (2 * _TK, _C), lambda t: (jnp.minimum(t, _B * _KT2 - 1) % _KT2, 0)
        ),
    ],
    out_specs=[
        pl.BlockSpec((1, 1, _HW), lambda t: (jnp.maximum(t // _KT2 - 1, 0), 0, 0)),
        pl.BlockSpec((1, 1), lambda t: (0, 0), memory_space=pltpu.SMEM),
    ],
    out_shape=[
        jax.ShapeDtypeStruct((_B, 1, _HW), jnp.int32),
        jax.ShapeDtypeStruct((1, 1), jnp.float32),
    ],
    scratch_shapes=[
        pltpu.VMEM((_TK, _HW), jnp.float32),
        pltpu.VMEM((_TK, 1), jnp.float32),
        pltpu.VMEM((1, _HW), jnp.float32),
        pltpu.VMEM((1, _HW), jnp.float32),
        pltpu.VMEM((1, _HW), jnp.int32),
    ],
    compiler_params=pltpu.CompilerParams(
        dimension_semantics=("arbitrary",),
    ),
)


@functools.cache
def _gather_rows_call():
    @functools.partial(
        pl.kernel,
        out_type=jax.ShapeDtypeStruct((_N, _C), jnp.float32),
        mesh=plsc.VectorSubcoreMesh(core_axis_name="c", subcore_axis_name="s"),
        scratch_types=[
            pltpu.VMEM((_RPW,), jnp.int32),
            pltpu.VMEM((_RPW, _C), jnp.float32),
            pltpu.SemaphoreType.DMA,
        ],
    )
    def _gather_rows(w_hbm, idx_hbm, z_hbm, idx_v, rows_v, sem):
        wid = lax.axis_index("s") * _NC + lax.axis_index("c")
        base = wid * _RPW
        pltpu.sync_copy(idx_hbm.at[pl.ds(base, _RPW)], idx_v)
        pltpu.async_copy(w_hbm.at[idx_v], rows_v, sem).wait()
        pltpu.sync_copy(rows_v, z_hbm.at[pl.ds(base, _RPW)])

    return _gather_rows


def kernel(x, W):
    B, C, H, Wd = x.shape
    xr = x.reshape(B, C, H * Wd)
    midx, loss_sum = _argmin_call(xr, W)
    idx_flat = midx.reshape(B * H * Wd)
    z = _gather_rows_call()(W, idx_flat)
    z_out = jnp.transpose(z.reshape(B, H, Wd, C), (0, 3, 1, 2))
    sequence = midx.reshape(B, H, Wd)
    loss = loss_sum[0, 0]
    return (z_out, sequence, loss, loss)
